# baseline (speedup 1.0000x reference)
import jax
import jax.numpy as jnp
import numpy as np
from jax import lax
from jax.experimental import pallas as pl
from jax.experimental.pallas import tpu as pltpu
from jax.experimental.pallas import tpu_sc as plsc

F32 = jnp.float32
I32 = jnp.int32

_N = 10000
_E = 320000
_HID = 128
_HEADS = 8
_C = 16
_NC = 2
_NS = 16
_L = 16
_NW = _NC * _NS
_EPW = _E // _NW
_CH = 32
_NCH = 314
_EPWP = _NCH * _CH
_PAD = _EPWP - _EPW
_G = _CH // _L
_OWN = 640
_DW = 8
_DR = 648
_DRX = 640

_BLK = 1000
_GRID = _N // _BLK


def _sc_edge_body(q_hbm, k_hbm, v_hbm, src_hbm, dst_hbm,
                  num_out, den_out,
                  srcv0, dstv0, drib0, qb0, kb0, vb0, eb0,
                  srcv1, dstv1, drib1, qb1, kb1, vb1, eb1,
                  sg0, sg1, sn0, sn1, sd0, sd1,
                  num_sh, den_sh):
    c = lax.axis_index("c")
    s = lax.axis_index("s")
    wid = c * _NS + s
    iota = lax.iota(I32, _L)
    zero16 = jnp.zeros((_L,), F32)
    sets = ((srcv0, dstv0, drib0, qb0, kb0, vb0, eb0, sg0, sn0, sd0),
            (srcv1, dstv1, drib1, qb1, kb1, vb1, eb1, sg1, sn1, sd1))

    def zb(i, carry):
        for j in range(_HID // _L):
            qb0[i, pl.ds(j * _L, _L)] = zero16
            eb0[i, pl.ds(j * _L, _L)] = zero16
            eb1[i, pl.ds(j * _L, _L)] = zero16
        return carry
    lax.fori_loop(0, _CH, zb, 0)
    r0 = s * _OWN
    ncp = jnp.where(s < _NS - 1, _OWN // _CH, 400 // _CH)

    def zcp(t, carry):
        off = pl.multiple_of(r0 + t * _CH, 8)
        pltpu.sync_copy(qb0, num_sh.at[pl.ds(off, _CH)])
        return carry
    lax.fori_loop(0, ncp, zcp, 0)

    @pl.when(s == _NS - 1)
    def _():
        pltpu.sync_copy(qb0.at[pl.ds(0, 24)], num_sh.at[pl.ds(9984, 24)])
    pltpu.sync_copy(qb0.at[pl.ds(0, 24)],
                    den_sh.at[pl.ds(s * 40, 24)])
    pltpu.sync_copy(qb0.at[pl.ds(0, 16)],
                    den_sh.at[pl.ds(s * 40 + 24, 16)])

    @pl.when(s == _NS - 1)
    def _():
        pltpu.sync_copy(qb0.at[pl.ds(0, 8)], den_sh.at[pl.ds(640, 8)])
    plsc.subcore_barrier()

    ebase = wid * _EPWP

    def load_and_fire(i, st):
        srcv, dstv, drib, qb, kb, vb, eb, sg, sn, sd = st
        base = pl.multiple_of(ebase + i * _CH, 8)
        pltpu.sync_copy(src_hbm.at[pl.ds(base, _CH)], srcv)
        pltpu.sync_copy(dst_hbm.at[pl.ds(base, _CH)], dstv)
        pltpu.async_copy(q_hbm.at[dstv], qb, sg)
        pltpu.async_copy(k_hbm.at[srcv], kb, sg)
        pltpu.async_copy(v_hbm.at[srcv], vb, sg)

    def wait_gathers(st):
        srcv, dstv, drib, qb, kb, vb, eb, sg, sn, sd = st
        pltpu.make_async_copy(q_hbm.at[dstv], qb, sg).wait()
        pltpu.make_async_copy(k_hbm.at[srcv], kb, sg).wait()
        pltpu.make_async_copy(v_hbm.at[srcv], vb, sg).wait()

    def compute(st):
        srcv, dstv, drib, qb, kb, vb, eb, sg, sn, sd = st

        def grp(g, carry):
            rows = g * _L + iota
            dvals = dstv[pl.ds(g * _L, _L)]
            drib[pl.ds(g * _L, _L)] = lax.shift_right_logical(dvals, 4)
            dlow8 = (dvals & 15) * 8
            for h in range(_HEADS):
                acc = jnp.zeros((_L,), F32)
                for cc in range(_C):
                    colv = jnp.full((_L,), h * _C + cc, I32)
                    acc = acc + (plsc.load_gather(qb, [rows, colv]) *
                                 plsc.load_gather(kb, [rows, colv]))
                ex = jnp.exp(acc)
                plsc.store_scatter(eb, [rows, dlow8 + h], ex)
                for cc in range(_C):
                    colv = jnp.full((_L,), h * _C + cc, I32)
                    vv = plsc.load_gather(vb, [rows, colv])
                    plsc.store_scatter(vb, [rows, colv], vv * ex)
            return carry
        lax.fori_loop(0, _G, grp, 0)

    def fire_scatters(st):
        srcv, dstv, drib, qb, kb, vb, eb, sg, sn, sd = st
        pltpu.async_copy(vb, num_sh.at[dstv], sn, add=True)
        pltpu.async_copy(eb, den_sh.at[drib], sd, add=True)

    def wait_scatters_and_clear(st):
        srcv, dstv, drib, qb, kb, vb, eb, sg, sn, sd = st
        pltpu.make_async_copy(vb, num_sh.at[dstv], sn).wait()
        pltpu.make_async_copy(eb, den_sh.at[drib], sd).wait()

        def grp_c(g, carry):
            rows = g * _L + iota
            dlow8 = (dstv[pl.ds(g * _L, _L)] & 15) * 8
            for h in range(_HEADS):
                plsc.store_scatter(eb, [rows, dlow8 + h], zero16)
            return carry
        lax.fori_loop(0, _G, grp_c, 0)

    load_and_fire(0, sets[0])

    def step(i2, carry):
        for b in range(2):
            i = i2 * 2 + b
            st = sets[b]
            other = sets[1 - b]
            wait_gathers(st)
            compute(st)
            fire_scatters(st)

            @pl.when(i >= 1)
            def _():
                wait_scatters_and_clear(other)

            @pl.when(i + 1 < _NCH)
            def _():
                load_and_fire(i + 1, other)
        return carry
    lax.fori_loop(0, _NCH // 2, step, 0)
    wait_scatters_and_clear(sets[1])

    plsc.subcore_barrier()

    def ocp(t, carry):
        off = pl.multiple_of(r0 + t * _CH, 8)
        pltpu.sync_copy(num_sh.at[pl.ds(off, _CH)],
                        num_out.at[c, pl.ds(off, _CH)])
        return carry
    lax.fori_loop(0, ncp, ocp, 0)

    @pl.when(s == _NS - 1)
    def _():
        pltpu.sync_copy(num_sh.at[pl.ds(9984, 16)],
                        num_out.at[c, pl.ds(9984, 16)])
    pltpu.sync_copy(den_sh.at[pl.ds(s * 40, 40)],
                    den_out.at[c, pl.ds(s * 40, 40)])


def _sc_edge(q, k, v, src_p, dst_p):
    mesh = plsc.VectorSubcoreMesh(core_axis_name="c", subcore_axis_name="s",
                                  num_cores=_NC, num_subcores=_NS)
    idx = pltpu.VMEM((_CH,), I32)
    row = pltpu.VMEM((_CH, _HID), F32)
    kern = pl.kernel(
        _sc_edge_body,
        out_type=(jax.ShapeDtypeStruct((_NC, _N, _HID), F32),
                  jax.ShapeDtypeStruct((_NC, _DRX, _HID), F32)),
        mesh=mesh,
        scratch_types=[
            idx, idx, idx, row, row, row, row,
            idx, idx, idx, row, row, row, row,
            pltpu.SemaphoreType.DMA, pltpu.SemaphoreType.DMA,
            pltpu.SemaphoreType.DMA, pltpu.SemaphoreType.DMA,
            pltpu.SemaphoreType.DMA, pltpu.SemaphoreType.DMA,
            pltpu.VMEM_SHARED((_N + 8, _HID), F32),
            pltpu.VMEM_SHARED((_DR, _HID), F32),
        ],
        compiler_params=pltpu.CompilerParams(needs_layout_passes=False),
    )
    num_pair, den_rows = kern(q, k, v, src_p, dst_p)
    den_pair = den_rows.reshape(_NC, _DRX * _C, _DW)[:, :_N]
    return num_pair, den_pair


def _pad_edges(src, dst):
    src2 = src.reshape(_NW, _EPW)
    dst2 = dst.reshape(_NW, _EPW)
    sp = jnp.zeros((_NW, _PAD), I32)
    dp = jnp.full((_NW, _PAD), _N, I32)
    return (jnp.concatenate([src2, sp], axis=1).reshape(-1),
            jnp.concatenate([dst2, dp], axis=1).reshape(-1))


def _gelu(x):
    return 0.5 * x * (1.0 + lax.erf(x * np.float32(1.0 / np.sqrt(2.0))))


def _dot(a, b):
    return jnp.dot(a, b, preferred_element_type=F32)


def _tca0_body(x_ref, winT, binr, wqT, bqr, wkT, bkr, wvT, bvr,
               h_ref, q_ref, k_ref, v_ref):
    h = _gelu(_dot(x_ref[...], winT[...]) + binr[...])
    h_ref[...] = h
    q_ref[...] = (_dot(h, wqT[...]) + bqr[...]) * 0.25
    k_ref[...] = _dot(h, wkT[...]) + bkr[...]
    v_ref[...] = _dot(h, wvT[...]) + bvr[...]


def _tcb_core(h, np_ref, dp_ref, exp_ref, wsT, bs, wbo, wbs, lng, lnb,
              w1T, b1, w2T, b2):
    num = np_ref[0] + np_ref[1]
    den = dp_ref[0] + dp_ref[1]
    den_e = _dot(den, exp_ref[...])
    out = num / (den_e + 1e-16)
    skip = _dot(h, wsT[...]) + bs[...]
    beta = jax.nn.sigmoid(
        jnp.sum(out * wbo[...] + skip * wbs[...], axis=1, keepdims=True))
    g = beta * skip + (1.0 - beta) * out + h
    mu = jnp.mean(g, axis=1, keepdims=True)
    gc = g - mu
    var = jnp.mean(gc * gc, axis=1, keepdims=True)
    hn = gc * lax.rsqrt(var + 1e-5) * lng[...] + lnb[...]
    f = _gelu(_dot(hn, w1T[...]) + b1[...])
    f = _dot(f, w2T[...]) + b2[...]
    return f + hn


def _tcb_mid_body(h_ref, np_ref, dp_ref, exp_ref,
                  wsT, bs, wbo, wbs, lng, lnb, w1T, b1, w2T, b2,
                  wqT, bq, wkT, bk, wvT, bv,
                  ho_ref, q_ref, k_ref, v_ref):
    h2 = _tcb_core(h_ref[...], np_ref, dp_ref, exp_ref, wsT, bs, wbo, wbs,
                   lng, lnb, w1T, b1, w2T, b2)
    ho_ref[...] = h2
    q_ref[...] = (_dot(h2, wqT[...]) + bq[...]) * 0.25
    k_ref[...] = _dot(h2, wkT[...]) + bk[...]
    v_ref[...] = _dot(h2, wvT[...]) + bv[...]


def _tcb_last_body(h_ref, np_ref, dp_ref, exp_ref,
                   wsT, bs, wbo, wbs, lng, lnb, w1T, b1, w2T, b2,
                   woT, bo, y_ref):
    h2 = _tcb_core(h_ref[...], np_ref, dp_ref, exp_ref, wsT, bs, wbo, wbs,
                   lng, lnb, w1T, b1, w2T, b2)
    y_ref[...] = _dot(h2, woT[...]) + bo[...]


_ROWS = pl.BlockSpec((_BLK, _HID), lambda i: (i, 0))
_ROWS4 = pl.BlockSpec((_BLK, 4 * _HID), lambda i: (i, 0))


def _wspec(shape):
    nd = len(shape)
    return pl.BlockSpec(shape, lambda i, nd=nd: (0,) * nd)


def _tca0(x, winT, binr, wqT, bqr, wkT, bkr, wvT, bvr):
    return pl.pallas_call(
        _tca0_body,
        grid=(_GRID,),
        in_specs=[_ROWS] + [_wspec(a.shape)
                            for a in (winT, binr, wqT, bqr, wkT, bkr, wvT, bvr)],
        out_specs=[_ROWS] * 4,
        out_shape=[jax.ShapeDtypeStruct((_N, _HID), F32)] * 4,
    )(x, winT, binr, wqT, bqr, wkT, bkr, wvT, bvr)


def _tcb_mid(h, num_pair, den_pair, expand, *ws):
    np_spec = pl.BlockSpec((_NC, _BLK, _HID), lambda i: (0, i, 0))
    dp_spec = pl.BlockSpec((_NC, _BLK, _DW), lambda i: (0, i, 0))
    return pl.pallas_call(
        _tcb_mid_body,
        grid=(_GRID,),
        in_specs=[_ROWS, np_spec, dp_spec, _wspec(expand.shape)]
                 + [_wspec(a.shape) for a in ws],
        out_specs=[_ROWS] * 4,
        out_shape=[jax.ShapeDtypeStruct((_N, _HID), F32)] * 4,
    )(h, num_pair, den_pair, expand, *ws)


def _tcb_last(h, num_pair, den_pair, expand, *ws):
    np_spec = pl.BlockSpec((_NC, _BLK, _HID), lambda i: (0, i, 0))
    dp_spec = pl.BlockSpec((_NC, _BLK, _DW), lambda i: (0, i, 0))
    return pl.pallas_call(
        _tcb_last_body,
        grid=(_GRID,),
        in_specs=[_ROWS, np_spec, dp_spec, _wspec(expand.shape)]
                 + [_wspec(a.shape) for a in ws],
        out_specs=_ROWS,
        out_shape=jax.ShapeDtypeStruct((_N, _HID), F32),
    )(h, num_pair, den_pair, expand, *ws)


def _row(b):
    return b.reshape(1, -1)


def kernel(x, edge_index, params):
    p = params
    src, dst = _pad_edges(edge_index[0], edge_index[1])
    layers = p['Wq'].shape[0]

    expand = np.zeros((_DW, _HID), np.float32)
    for h in range(_HEADS):
        expand[h, h * _C:(h + 1) * _C] = 1.0
    expand = jnp.asarray(expand)

    def qkvw(i):
        return (p['Wq'][i].T, _row(p['bq'][i]), p['Wk'][i].T, _row(p['bk'][i]),
                p['Wv'][i].T, _row(p['bv'][i]))

    def layerw(i):
        wb = p['Wbeta'][i][0]
        wbo = _row(wb[:_HID] + wb[2 * _HID:])
        wbs = _row(wb[_HID:2 * _HID] - wb[2 * _HID:])
        return (p['Wskip'][i].T, _row(p['bskip'][i]), wbo, wbs,
                _row(p['ln_g'][i]), _row(p['ln_b'][i]),
                p['W1'][i].T, _row(p['b1'][i]), p['W2'][i].T, _row(p['b2'][i]))

    h, q, k, v = _tca0(x, p['Win'].T, _row(p['bin']), *qkvw(0))
    for i in range(layers):
        num_pair, den_pair = _sc_edge(q, k, v, src, dst)
        if i < layers - 1:
            h, q, k, v = _tcb_mid(h, num_pair, den_pair, expand,
                                  *layerw(i), *qkvw(i + 1))
        else:
            y = _tcb_last(h, num_pair, den_pair, expand,
                          *layerw(i), p['Wout'].T, _row(p['bout']))
    return y


# speedup vs baseline: 1.0033x; 1.0033x over previous
import jax
import jax.numpy as jnp
import numpy as np
from jax import lax
from jax.experimental import pallas as pl
from jax.experimental.pallas import tpu as pltpu
from jax.experimental.pallas import tpu_sc as plsc

F32 = jnp.float32
I32 = jnp.int32

_N = 10000
_E = 320000
_HID = 128
_HEADS = 8
_C = 16
_NC = 2
_NS = 16
_L = 16
_NW = _NC * _NS
_EPW = _E // _NW
_CH = 32
_NCH = 314
_EPWP = _NCH * _CH
_PAD = _EPWP - _EPW
_G = _CH // _L
_OWN = 640
_DW = 8
_DR = 648
_DRX = 640

_BLK = 1000
_GRID = _N // _BLK


def _sc_edge_body(q_hbm, kv_hbm, src_hbm, dst_hbm,
                  num_out, den_out,
                  srcv0, dstv0, drib0, qb0, kvb0, mb0, eb0,
                  srcv1, dstv1, drib1, qb1, kvb1, mb1, eb1,
                  sg0, sg1, sn0, sn1, sd0, sd1,
                  num_sh, den_sh):
    c = lax.axis_index("c")
    s = lax.axis_index("s")
    wid = c * _NS + s
    iota = lax.iota(I32, _L)
    zero16 = jnp.zeros((_L,), F32)
    sets = ((srcv0, dstv0, drib0, qb0, kvb0, mb0, eb0, sg0, sn0, sd0),
            (srcv1, dstv1, drib1, qb1, kvb1, mb1, eb1, sg1, sn1, sd1))

    def zb(i, carry):
        for j in range(_HID // _L):
            qb0[i, pl.ds(j * _L, _L)] = zero16
            eb0[i, pl.ds(j * _L, _L)] = zero16
            eb1[i, pl.ds(j * _L, _L)] = zero16
        return carry
    lax.fori_loop(0, _CH, zb, 0)
    r0 = s * _OWN
    ncp = jnp.where(s < _NS - 1, _OWN // _CH, 400 // _CH)

    def zcp(t, carry):
        off = pl.multiple_of(r0 + t * _CH, 8)
        pltpu.sync_copy(qb0, num_sh.at[pl.ds(off, _CH)])
        return carry
    lax.fori_loop(0, ncp, zcp, 0)

    @pl.when(s == _NS - 1)
    def _():
        pltpu.sync_copy(qb0.at[pl.ds(0, 24)], num_sh.at[pl.ds(9984, 24)])
    pltpu.sync_copy(qb0.at[pl.ds(0, 24)],
                    den_sh.at[pl.ds(s * 40, 24)])
    pltpu.sync_copy(qb0.at[pl.ds(0, 16)],
                    den_sh.at[pl.ds(s * 40 + 24, 16)])

    @pl.when(s == _NS - 1)
    def _():
        pltpu.sync_copy(qb0.at[pl.ds(0, 8)], den_sh.at[pl.ds(640, 8)])
    plsc.subcore_barrier()

    ebase = wid * _EPWP

    def load_and_fire(i, st):
        srcv, dstv, drib, qb, kvb, mb, eb, sg, sn, sd = st
        base = pl.multiple_of(ebase + i * _CH, 8)
        pltpu.sync_copy(src_hbm.at[pl.ds(base, _CH)], srcv)
        pltpu.sync_copy(dst_hbm.at[pl.ds(base, _CH)], dstv)
        pltpu.async_copy(q_hbm.at[dstv], qb, sg)
        pltpu.async_copy(kv_hbm.at[srcv], kvb, sg)

    def wait_gathers(st):
        srcv, dstv, drib, qb, kvb, mb, eb, sg, sn, sd = st
        pltpu.make_async_copy(q_hbm.at[dstv], qb, sg).wait()
        pltpu.make_async_copy(kv_hbm.at[srcv], kvb, sg).wait()

    def compute(st):
        srcv, dstv, drib, qb, kvb, mb, eb, sg, sn, sd = st

        def grp(g, carry):
            rows = g * _L + iota
            dvals = dstv[pl.ds(g * _L, _L)]
            drib[pl.ds(g * _L, _L)] = lax.shift_right_logical(dvals, 4)
            dlow8 = (dvals & 15) * 8
            for h in range(_HEADS):
                acc = jnp.zeros((_L,), F32)
                for cc in range(_C):
                    colv = jnp.full((_L,), h * _C + cc, I32)
                    acc = acc + (plsc.load_gather(qb, [rows, colv]) *
                                 plsc.load_gather(kvb, [rows, colv]))
                ex = jnp.exp(acc)
                plsc.store_scatter(eb, [rows, dlow8 + h], ex)
                for cc in range(_C):
                    colv = jnp.full((_L,), h * _C + cc, I32)
                    vcol = jnp.full((_L,), _HID + h * _C + cc, I32)
                    vv = plsc.load_gather(kvb, [rows, vcol])
                    plsc.store_scatter(mb, [rows, colv], vv * ex)
            return carry
        lax.fori_loop(0, _G, grp, 0)

    def fire_scatters(st):
        srcv, dstv, drib, qb, kvb, mb, eb, sg, sn, sd = st
        pltpu.async_copy(mb, num_sh.at[dstv], sn, add=True)
        pltpu.async_copy(eb, den_sh.at[drib], sd, add=True)

    def wait_scatters_and_clear(st):
        srcv, dstv, drib, qb, kvb, mb, eb, sg, sn, sd = st
        pltpu.make_async_copy(mb, num_sh.at[dstv], sn).wait()
        pltpu.make_async_copy(eb, den_sh.at[drib], sd).wait()

        def grp_c(g, carry):
            rows = g * _L + iota
            dlow8 = (dstv[pl.ds(g * _L, _L)] & 15) * 8
            for h in range(_HEADS):
                plsc.store_scatter(eb, [rows, dlow8 + h], zero16)
            return carry
        lax.fori_loop(0, _G, grp_c, 0)

    load_and_fire(0, sets[0])

    def step(i2, carry):
        for b in range(2):
            i = i2 * 2 + b
            st = sets[b]
            other = sets[1 - b]
            wait_gathers(st)
            compute(st)
            fire_scatters(st)

            @pl.when(i >= 1)
            def _():
                wait_scatters_and_clear(other)

            @pl.when(i + 1 < _NCH)
            def _():
                load_and_fire(i + 1, other)
        return carry
    lax.fori_loop(0, _NCH // 2, step, 0)
    wait_scatters_and_clear(sets[1])

    plsc.subcore_barrier()

    def ocp(t, carry):
        off = pl.multiple_of(r0 + t * _CH, 8)
        pltpu.sync_copy(num_sh.at[pl.ds(off, _CH)],
                        num_out.at[c, pl.ds(off, _CH)])
        return carry
    lax.fori_loop(0, ncp, ocp, 0)

    @pl.when(s == _NS - 1)
    def _():
        pltpu.sync_copy(num_sh.at[pl.ds(9984, 16)],
                        num_out.at[c, pl.ds(9984, 16)])
    pltpu.sync_copy(den_sh.at[pl.ds(s * 40, 40)],
                    den_out.at[c, pl.ds(s * 40, 40)])


def _sc_edge(q, kv, src_p, dst_p):
    mesh = plsc.VectorSubcoreMesh(core_axis_name="c", subcore_axis_name="s",
                                  num_cores=_NC, num_subcores=_NS)
    idx = pltpu.VMEM((_CH,), I32)
    row = pltpu.VMEM((_CH, _HID), F32)
    row2 = pltpu.VMEM((_CH, 2 * _HID), F32)
    kern = pl.kernel(
        _sc_edge_body,
        out_type=(jax.ShapeDtypeStruct((_NC, _N, _HID), F32),
                  jax.ShapeDtypeStruct((_NC, _DRX, _HID), F32)),
        mesh=mesh,
        scratch_types=[
            idx, idx, idx, row, row2, row, row,
            idx, idx, idx, row, row2, row, row,
            pltpu.SemaphoreType.DMA, pltpu.SemaphoreType.DMA,
            pltpu.SemaphoreType.DMA, pltpu.SemaphoreType.DMA,
            pltpu.SemaphoreType.DMA, pltpu.SemaphoreType.DMA,
            pltpu.VMEM_SHARED((_N + 8, _HID), F32),
            pltpu.VMEM_SHARED((_DR, _HID), F32),
        ],
        compiler_params=pltpu.CompilerParams(needs_layout_passes=False),
    )
    num_pair, den_rows = kern(q, kv, src_p, dst_p)
    den_pair = den_rows.reshape(_NC, _DRX * _C, _DW)[:, :_N]
    return num_pair, den_pair


def _pad_edges(src, dst):
    src2 = src.reshape(_NW, _EPW)
    dst2 = dst.reshape(_NW, _EPW)
    sp = jnp.zeros((_NW, _PAD), I32)
    dp = jnp.full((_NW, _PAD), _N, I32)
    return (jnp.concatenate([src2, sp], axis=1).reshape(-1),
            jnp.concatenate([dst2, dp], axis=1).reshape(-1))


def _gelu(x):
    return 0.5 * x * (1.0 + lax.erf(x * np.float32(1.0 / np.sqrt(2.0))))


def _dot(a, b):
    return jnp.dot(a, b, preferred_element_type=F32)


def _tca0_body(x_ref, winT, binr, wqT, bqr, wkT, bkr, wvT, bvr,
               h_ref, q_ref, kv_ref):
    h = _gelu(_dot(x_ref[...], winT[...]) + binr[...])
    h_ref[...] = h
    q_ref[...] = (_dot(h, wqT[...]) + bqr[...]) * 0.25
    kv_ref[...] = jnp.concatenate(
        [_dot(h, wkT[...]) + bkr[...], _dot(h, wvT[...]) + bvr[...]], axis=1)


def _tcb_core(h, np_ref, dp_ref, exp_ref, wsT, bs, wbo, wbs, lng, lnb,
              w1T, b1, w2T, b2):
    num = np_ref[0] + np_ref[1]
    den = dp_ref[0] + dp_ref[1]
    den_e = _dot(den, exp_ref[...])
    out = num / (den_e + 1e-16)
    skip = _dot(h, wsT[...]) + bs[...]
    beta = jax.nn.sigmoid(
        jnp.sum(out * wbo[...] + skip * wbs[...], axis=1, keepdims=True))
    g = beta * skip + (1.0 - beta) * out + h
    mu = jnp.mean(g, axis=1, keepdims=True)
    gc = g - mu
    var = jnp.mean(gc * gc, axis=1, keepdims=True)
    hn = gc * lax.rsqrt(var + 1e-5) * lng[...] + lnb[...]
    f = _gelu(_dot(hn, w1T[...]) + b1[...])
    f = _dot(f, w2T[...]) + b2[...]
    return f + hn


def _tcb_mid_body(h_ref, np_ref, dp_ref, exp_ref,
                  wsT, bs, wbo, wbs, lng, lnb, w1T, b1, w2T, b2,
                  wqT, bq, wkT, bk, wvT, bv,
                  ho_ref, q_ref, kv_ref):
    h2 = _tcb_core(h_ref[...], np_ref, dp_ref, exp_ref, wsT, bs, wbo, wbs,
                   lng, lnb, w1T, b1, w2T, b2)
    ho_ref[...] = h2
    q_ref[...] = (_dot(h2, wqT[...]) + bq[...]) * 0.25
    kv_ref[...] = jnp.concatenate(
        [_dot(h2, wkT[...]) + bk[...], _dot(h2, wvT[...]) + bv[...]], axis=1)


def _tcb_last_body(h_ref, np_ref, dp_ref, exp_ref,
                   wsT, bs, wbo, wbs, lng, lnb, w1T, b1, w2T, b2,
                   woT, bo, y_ref):
    h2 = _tcb_core(h_ref[...], np_ref, dp_ref, exp_ref, wsT, bs, wbo, wbs,
                   lng, lnb, w1T, b1, w2T, b2)
    y_ref[...] = _dot(h2, woT[...]) + bo[...]


_ROWS = pl.BlockSpec((_BLK, _HID), lambda i: (i, 0))
_ROWS4 = pl.BlockSpec((_BLK, 4 * _HID), lambda i: (i, 0))


def _wspec(shape):
    nd = len(shape)
    return pl.BlockSpec(shape, lambda i, nd=nd: (0,) * nd)


_ROWS2 = pl.BlockSpec((_BLK, 2 * _HID), lambda i: (i, 0))


def _tca0(x, winT, binr, wqT, bqr, wkT, bkr, wvT, bvr):
    return pl.pallas_call(
        _tca0_body,
        grid=(_GRID,),
        in_specs=[_ROWS] + [_wspec(a.shape)
                            for a in (winT, binr, wqT, bqr, wkT, bkr, wvT, bvr)],
        out_specs=[_ROWS, _ROWS, _ROWS2],
        out_shape=[jax.ShapeDtypeStruct((_N, _HID), F32)] * 2
        + [jax.ShapeDtypeStruct((_N, 2 * _HID), F32)],
    )(x, winT, binr, wqT, bqr, wkT, bkr, wvT, bvr)


def _tcb_mid(h, num_pair, den_pair, expand, *ws):
    np_spec = pl.BlockSpec((_NC, _BLK, _HID), lambda i: (0, i, 0))
    dp_spec = pl.BlockSpec((_NC, _BLK, _DW), lambda i: (0, i, 0))
    return pl.pallas_call(
        _tcb_mid_body,
        grid=(_GRID,),
        in_specs=[_ROWS, np_spec, dp_spec, _wspec(expand.shape)]
                 + [_wspec(a.shape) for a in ws],
        out_specs=[_ROWS, _ROWS, _ROWS2],
        out_shape=[jax.ShapeDtypeStruct((_N, _HID), F32)] * 2
        + [jax.ShapeDtypeStruct((_N, 2 * _HID), F32)],
    )(h, num_pair, den_pair, expand, *ws)


def _tcb_last(h, num_pair, den_pair, expand, *ws):
    np_spec = pl.BlockSpec((_NC, _BLK, _HID), lambda i: (0, i, 0))
    dp_spec = pl.BlockSpec((_NC, _BLK, _DW), lambda i: (0, i, 0))
    return pl.pallas_call(
        _tcb_last_body,
        grid=(_GRID,),
        in_specs=[_ROWS, np_spec, dp_spec, _wspec(expand.shape)]
                 + [_wspec(a.shape) for a in ws],
        out_specs=_ROWS,
        out_shape=jax.ShapeDtypeStruct((_N, _HID), F32),
    )(h, num_pair, den_pair, expand, *ws)


def _row(b):
    return b.reshape(1, -1)


def kernel(x, edge_index, params):
    p = params
    src, dst = _pad_edges(edge_index[0], edge_index[1])
    layers = p['Wq'].shape[0]

    expand = np.zeros((_DW, _HID), np.float32)
    for h in range(_HEADS):
        expand[h, h * _C:(h + 1) * _C] = 1.0
    expand = jnp.asarray(expand)

    def qkvw(i):
        return (p['Wq'][i].T, _row(p['bq'][i]), p['Wk'][i].T, _row(p['bk'][i]),
                p['Wv'][i].T, _row(p['bv'][i]))

    def layerw(i):
        wb = p['Wbeta'][i][0]
        wbo = _row(wb[:_HID] + wb[2 * _HID:])
        wbs = _row(wb[_HID:2 * _HID] - wb[2 * _HID:])
        return (p['Wskip'][i].T, _row(p['bskip'][i]), wbo, wbs,
                _row(p['ln_g'][i]), _row(p['ln_b'][i]),
                p['W1'][i].T, _row(p['b1'][i]), p['W2'][i].T, _row(p['b2'][i]))

    h, q, kv = _tca0(x, p['Win'].T, _row(p['bin']), *qkvw(0))
    for i in range(layers):
        num_pair, den_pair = _sc_edge(q, kv, src, dst)
        if i < layers - 1:
            h, q, kv = _tcb_mid(h, num_pair, den_pair, expand,
                                *layerw(i), *qkvw(i + 1))
        else:
            y = _tcb_last(h, num_pair, den_pair, expand,
                          *layerw(i), p['Wout'].T, _row(p['bout']))
    return y


# speedup vs baseline: 3.5976x; 3.5859x over previous
import functools

import jax
import jax.numpy as jnp
import numpy as np
from jax import lax
from jax.experimental import pallas as pl
from jax.experimental.pallas import tpu as pltpu
from jax.experimental.pallas import tpu_sc as plsc

F32 = jnp.float32
I32 = jnp.int32

_N = 10000
_E = 320000
_HID = 128
_HEADS = 8
_C = 16

_NC = 2
_NS = 16
_L = 16
_NW = _NC * _NS
_EPW = _E // _NW
_CH = 80
_NCH = _EPW // _CH
_G = _CH // _L
_OWN = 640
_DW = 8
_DR = 640

_BLK = 1000
_GRID = _N // _BLK


def _sc_edge_body(q_hbm, k_hbm, v_hbm, src_hbm, dst_hbm,
                  num_out, den_out,
                  srcv, dstv, drib, qb, kb, eb, num_sh, den_sh,
                  s1, s2):
    c = lax.axis_index("c")
    s = lax.axis_index("s")
    wid = c * _NS + s
    iota = lax.iota(I32, _L)
    zero16 = jnp.zeros((_L,), F32)

    def zero_body(i, carry):
        for j in range(_HID // _L):
            qb[i, pl.ds(j * _L, _L)] = zero16
            eb[i, pl.ds(j * _L, _L)] = zero16
        return carry
    lax.fori_loop(0, _CH, zero_body, 0)
    r0 = s * _OWN
    ncp = jnp.where(s < _NS - 1, _OWN // _CH, (_N - (_NS - 1) * _OWN) // _CH)

    def zcp(t, carry):
        off = pl.multiple_of(r0 + t * _CH, 8)
        pltpu.sync_copy(qb, num_sh.at[pl.ds(off, _CH)])
        return carry
    lax.fori_loop(0, ncp, zcp, 0)
    pltpu.sync_copy(eb.at[pl.ds(0, _DR // _NS)],
                    den_sh.at[pl.ds(s * (_DR // _NS), _DR // _NS)])
    plsc.subcore_barrier()

    def chunk(i, carry):
        base = pl.multiple_of(wid * _EPW + i * _CH, 8)
        pltpu.sync_copy(src_hbm.at[pl.ds(base, _CH)], srcv)
        pltpu.sync_copy(dst_hbm.at[pl.ds(base, _CH)], dstv)
        cq = pltpu.async_copy(q_hbm.at[dstv], qb, s1)
        ck = pltpu.async_copy(k_hbm.at[srcv], kb, s2)
        cq.wait()
        ck.wait()

        def grp_a(g, carry2):
            dvals = dstv[pl.ds(g * _L, _L)]
            drib[pl.ds(g * _L, _L)] = lax.shift_right_logical(dvals, 4)
            dlow8v = (dvals & 15) * 8
            for j in range(_L):
                e = g * _L + j
                jv = jnp.full((_L,), j, I32)
                dlow8 = dlow8v.at[jv].get(mode='promise_in_bounds')
                vlog = zero16
                for h in range(_HEADS):
                    p = qb[e, pl.ds(h * _C, _C)] * kb[e, pl.ds(h * _C, _C)]
                    lg = jnp.sum(p)
                    vlog = jnp.where(iota == h, jnp.full((_L,), lg), vlog)
                vex = jnp.exp(vlog)
                plsc.store_scatter(eb, [jnp.full((_L,), e, I32),
                                        dlow8 + (iota & 7)], vex,
                                   mask=iota < _HEADS)
            return carry2
        lax.fori_loop(0, _G, grp_a, 0)

        cv = pltpu.async_copy(v_hbm.at[srcv], qb, s1)
        cv.wait()

        def grp_b(g, carry2):
            dlow8v = (dstv[pl.ds(g * _L, _L)] & 15) * 8
            for j in range(_L):
                e = g * _L + j
                jv = jnp.full((_L,), j, I32)
                dlow8 = dlow8v.at[jv].get(mode='promise_in_bounds')
                exv8 = plsc.load_gather(eb, [jnp.full((_L,), e, I32),
                                             dlow8 + (iota & 7)])
                for h in range(_HEADS):
                    exb = exv8.at[jnp.full((_L,), h, I32)].get(
                        mode='promise_in_bounds')
                    vrow = qb[e, pl.ds(h * _C, _C)]
                    qb[e, pl.ds(h * _C, _C)] = vrow * exb
            return carry2
        lax.fori_loop(0, _G, grp_b, 0)

        pltpu.sync_copy(qb, num_sh.at[dstv], add=True)
        pltpu.sync_copy(eb, den_sh.at[drib], add=True)

        def grp_c(g, carry2):
            dlow8v = (dstv[pl.ds(g * _L, _L)] & 15) * 8
            for j in range(_L):
                e = g * _L + j
                jv = jnp.full((_L,), j, I32)
                dlow8 = dlow8v.at[jv].get(mode='promise_in_bounds')
                plsc.store_scatter(eb, [jnp.full((_L,), e, I32),
                                        dlow8 + (iota & 7)], zero16,
                                   mask=iota < _HEADS)
            return carry2
        lax.fori_loop(0, _G, grp_c, 0)
        return carry
    lax.fori_loop(0, _NCH, chunk, 0)

    plsc.subcore_barrier()

    def ocp(t, carry):
        off = pl.multiple_of(r0 + t * _CH, 8)
        pltpu.sync_copy(num_sh.at[pl.ds(off, _CH)],
                        num_out.at[c, pl.ds(off, _CH)])
        return carry
    lax.fori_loop(0, ncp, ocp, 0)
    pltpu.sync_copy(den_sh.at[pl.ds(s * (_DR // _NS), _DR // _NS)],
                    den_out.at[c, pl.ds(s * (_DR // _NS), _DR // _NS)])


def _sc_edge(q, k, v, src, dst):
    mesh = plsc.VectorSubcoreMesh(core_axis_name="c", subcore_axis_name="s",
                                  num_cores=_NC, num_subcores=_NS)
    kern = pl.kernel(
        _sc_edge_body,
        out_type=(jax.ShapeDtypeStruct((_NC, _N, _HID), F32),
                  jax.ShapeDtypeStruct((_NC, _DR, _HID), F32)),
        mesh=mesh,
        scratch_types=[
            pltpu.VMEM((_CH,), I32),
            pltpu.VMEM((_CH,), I32),
            pltpu.VMEM((_CH,), I32),
            pltpu.VMEM((_CH, _HID), F32),
            pltpu.VMEM((_CH, _HID), F32),
            pltpu.VMEM((_CH, _HID), F32),
            pltpu.VMEM_SHARED((_N, _HID), F32),
            pltpu.VMEM_SHARED((_DR, _HID), F32),
            pltpu.SemaphoreType.DMA,
            pltpu.SemaphoreType.DMA,
        ],
        compiler_params=pltpu.CompilerParams(needs_layout_passes=False),
    )
    num_pair, den_rows = kern(q, k, v, src, dst)
    den_pair = den_rows.reshape(_NC, _DR * _C, _DW)[:, :_N]
    return num_pair, den_pair


def _gelu(x):
    return 0.5 * x * (1.0 + lax.erf(x * np.float32(1.0 / np.sqrt(2.0))))


def _dot(a, b):
    return jnp.dot(a, b, preferred_element_type=F32)


def _tca0_body(x_ref, winT, binr, wqT, bqr, wkT, bkr, wvT, bvr,
               h_ref, q_ref, k_ref, v_ref):
    h = _gelu(_dot(x_ref[...], winT[...]) + binr[...])
    h_ref[...] = h
    q_ref[...] = (_dot(h, wqT[...]) + bqr[...]) * 0.25
    k_ref[...] = _dot(h, wkT[...]) + bkr[...]
    v_ref[...] = _dot(h, wvT[...]) + bvr[...]


def _tcb_core(h, np_ref, dp_ref, exp_ref, wsT, bs, wbo, wbs, lng, lnb,
              w1T, b1, w2T, b2):
    num = np_ref[0] + np_ref[1]
    den = dp_ref[0] + dp_ref[1]
    den_e = _dot(den, exp_ref[...])
    out = num / (den_e + 1e-16)
    skip = _dot(h, wsT[...]) + bs[...]
    beta = jax.nn.sigmoid(
        jnp.sum(out * wbo[...] + skip * wbs[...], axis=1, keepdims=True))
    g = beta * skip + (1.0 - beta) * out + h
    mu = jnp.mean(g, axis=1, keepdims=True)
    gc = g - mu
    var = jnp.mean(gc * gc, axis=1, keepdims=True)
    hn = gc * lax.rsqrt(var + 1e-5) * lng[...] + lnb[...]
    f = _gelu(_dot(hn, w1T[...]) + b1[...])
    f = _dot(f, w2T[...]) + b2[...]
    return f + hn


def _tcb_mid_body(h_ref, np_ref, dp_ref, exp_ref,
                  wsT, bs, wbo, wbs, lng, lnb, w1T, b1, w2T, b2,
                  wqT, bq, wkT, bk, wvT, bv,
                  ho_ref, q_ref, k_ref, v_ref):
    h2 = _tcb_core(h_ref[...], np_ref, dp_ref, exp_ref, wsT, bs, wbo, wbs,
                   lng, lnb, w1T, b1, w2T, b2)
    ho_ref[...] = h2
    q_ref[...] = (_dot(h2, wqT[...]) + bq[...]) * 0.25
    k_ref[...] = _dot(h2, wkT[...]) + bk[...]
    v_ref[...] = _dot(h2, wvT[...]) + bv[...]


def _tcb_last_body(h_ref, np_ref, dp_ref, exp_ref,
                   wsT, bs, wbo, wbs, lng, lnb, w1T, b1, w2T, b2,
                   woT, bo, y_ref):
    h2 = _tcb_core(h_ref[...], np_ref, dp_ref, exp_ref, wsT, bs, wbo, wbs,
                   lng, lnb, w1T, b1, w2T, b2)
    y_ref[...] = _dot(h2, woT[...]) + bo[...]


_ROWS = pl.BlockSpec((_BLK, _HID), lambda i: (i, 0))
_ROWS4 = pl.BlockSpec((_BLK, 4 * _HID), lambda i: (i, 0))


def _wspec(shape):
    nd = len(shape)
    return pl.BlockSpec(shape, lambda i, nd=nd: (0,) * nd)


def _tca0(x, winT, binr, wqT, bqr, wkT, bkr, wvT, bvr):
    return pl.pallas_call(
        _tca0_body,
        grid=(_GRID,),
        in_specs=[_ROWS] + [_wspec(a.shape)
                            for a in (winT, binr, wqT, bqr, wkT, bkr, wvT, bvr)],
        out_specs=[_ROWS] * 4,
        out_shape=[jax.ShapeDtypeStruct((_N, _HID), F32)] * 4,
    )(x, winT, binr, wqT, bqr, wkT, bkr, wvT, bvr)


def _tcb_mid(h, num_pair, den_pair, expand, *ws):
    np_spec = pl.BlockSpec((_NC, _BLK, _HID), lambda i: (0, i, 0))
    dp_spec = pl.BlockSpec((_NC, _BLK, _DW), lambda i: (0, i, 0))
    return pl.pallas_call(
        _tcb_mid_body,
        grid=(_GRID,),
        in_specs=[_ROWS, np_spec, dp_spec, _wspec(expand.shape)]
                 + [_wspec(a.shape) for a in ws],
        out_specs=[_ROWS] * 4,
        out_shape=[jax.ShapeDtypeStruct((_N, _HID), F32)] * 4,
    )(h, num_pair, den_pair, expand, *ws)


def _tcb_last(h, num_pair, den_pair, expand, *ws):
    np_spec = pl.BlockSpec((_NC, _BLK, _HID), lambda i: (0, i, 0))
    dp_spec = pl.BlockSpec((_NC, _BLK, _DW), lambda i: (0, i, 0))
    return pl.pallas_call(
        _tcb_last_body,
        grid=(_GRID,),
        in_specs=[_ROWS, np_spec, dp_spec, _wspec(expand.shape)]
                 + [_wspec(a.shape) for a in ws],
        out_specs=_ROWS,
        out_shape=jax.ShapeDtypeStruct((_N, _HID), F32),
    )(h, num_pair, den_pair, expand, *ws)


def _row(b):
    return b.reshape(1, -1)


def kernel(x, edge_index, params):
    p = params
    src = edge_index[0]
    dst = edge_index[1]
    layers = p['Wq'].shape[0]

    expand = np.zeros((_DW, _HID), np.float32)
    for h in range(_HEADS):
        expand[h, h * _C:(h + 1) * _C] = 1.0
    expand = jnp.asarray(expand)

    def qkvw(i):
        return (p['Wq'][i].T, _row(p['bq'][i]), p['Wk'][i].T, _row(p['bk'][i]),
                p['Wv'][i].T, _row(p['bv'][i]))

    def layerw(i):
        wb = p['Wbeta'][i][0]
        wbo = _row(wb[:_HID] + wb[2 * _HID:])
        wbs = _row(wb[_HID:2 * _HID] - wb[2 * _HID:])
        return (p['Wskip'][i].T, _row(p['bskip'][i]), wbo, wbs,
                _row(p['ln_g'][i]), _row(p['ln_b'][i]),
                p['W1'][i].T, _row(p['b1'][i]), p['W2'][i].T, _row(p['b2'][i]))

    h, q, k, v = _tca0(x, p['Win'].T, _row(p['bin']), *qkvw(0))
    for i in range(layers):
        num_pair, den_pair = _sc_edge(q, k, v, src, dst)
        if i < layers - 1:
            h, q, k, v = _tcb_mid(h, num_pair, den_pair, expand,
                                  *layerw(i), *qkvw(i + 1))
        else:
            y = _tcb_last(h, num_pair, den_pair, expand,
                          *layerw(i), p['Wout'].T, _row(p['bout']))
    return y
